# baseline (speedup 1.0000x reference)
import functools

import jax
import jax.numpy as jnp
from jax import lax
from jax.experimental import pallas as pl
from jax.experimental.pallas import tpu as pltpu
from jax.experimental.pallas import tpu_sc as plsc

B, S, D = 1, 2048, 768
H, KVH, DH = 12, 4, 64
E, TOPK, FF = 8, 2, 3072
THETA = 500000.0
CLIP = 8.0
REP = H // KVH

BT = 256
BQ = 512
BK = 512

NA = S * TOPK
CAP = S
BTC = 128
CAPB = CAP // BTC
BF = 1536
FFB = FF // BF
RCH = 512


def _dot_t(a, b):
    return jax.lax.dot_general(a, b, (((1,), (1,)), ((), ())),
                               preferred_element_type=jnp.float32)


def _qkv_kernel(x_ref, w_ref, wqkv_ref, cos_ref, sin_ref, o_ref):
    x = x_ref[...]
    mu = jnp.mean(x, axis=1, keepdims=True)
    xc = x - mu
    var = jnp.mean(xc * xc, axis=1, keepdims=True)
    xn = xc * jax.lax.rsqrt(var + 1e-5) * w_ref[...]
    qkv = _dot_t(xn, wqkv_ref[...])
    qkv = jnp.clip(qkv, -CLIP, CLIP)
    c = cos_ref[:, :DH // 2]
    s = sin_ref[:, :DH // 2]
    pieces = []
    for h in range(H + KVH):
        base = h * DH
        a = qkv[:, base:base + DH // 2]
        b = qkv[:, base + DH // 2:base + DH]
        pieces.append(a * c - b * s)
        pieces.append(b * c + a * s)
    pieces.append(qkv[:, (H + KVH) * DH:])
    o_ref[...] = jnp.concatenate(pieces, axis=1)


def _qkv_proj(x, ln1_w, wqkv, cos, sin):
    grid = (S // BT,)
    return pl.pallas_call(
        _qkv_kernel,
        grid=grid,
        in_specs=[
            pl.BlockSpec((BT, D), lambda i: (i, 0)),
            pl.BlockSpec((1, D), lambda i: (0, 0)),
            pl.BlockSpec(((H + 2 * KVH) * DH, D), lambda i: (0, 0)),
            pl.BlockSpec((BT, DH), lambda i: (i, 0)),
            pl.BlockSpec((BT, DH), lambda i: (i, 0)),
        ],
        out_specs=pl.BlockSpec((BT, (H + 2 * KVH) * DH), lambda i: (i, 0)),
        out_shape=jax.ShapeDtypeStruct((S, (H + 2 * KVH) * DH), jnp.float32),
    )(x, ln1_w.reshape(1, D), wqkv, cos, sin)


def _attn_kernel(q_ref, k_ref, v_ref, o_ref, s_ref):
    i = pl.program_id(1)
    nb = i + 1
    q = q_ref[0].reshape(REP * BQ, DH)
    scale = 1.0 / jnp.sqrt(jnp.float32(DH))

    def body1(j, m):
        k = k_ref[0, pl.ds(j * BK, BK), :]
        sc = _dot_t(q, k) * scale
        riota = jax.lax.broadcasted_iota(jnp.int32, (REP * BQ, BK), 0)
        qpos = i * BQ + (riota & (BQ - 1))
        kpos = j * BK + jax.lax.broadcasted_iota(jnp.int32, (REP * BQ, BK), 1)
        sc = jnp.where(qpos >= kpos, sc, -1e9)
        s_ref[:, pl.ds(j * BK, BK)] = sc
        return jnp.maximum(m, jnp.max(sc, axis=1, keepdims=True))

    m = jax.lax.fori_loop(0, nb, body1,
                          jnp.full((REP * BQ, 1), -jnp.inf, jnp.float32))

    def body2(j, carry):
        l, acc = carry
        pj = jnp.exp(s_ref[:, pl.ds(j * BK, BK)] - m)
        v = v_ref[0, pl.ds(j * BK, BK), :]
        return (l + jnp.sum(pj, axis=1, keepdims=True),
                acc + jnp.dot(pj, v, preferred_element_type=jnp.float32))

    l, acc = jax.lax.fori_loop(
        0, nb, body2,
        (jnp.zeros((REP * BQ, 1), jnp.float32),
         jnp.zeros((REP * BQ, DH), jnp.float32)))
    o_ref[0] = (acc / l).reshape(REP, BQ, DH)


def _attention(q4, k3, v3):
    grid = (KVH, S // BQ)
    return pl.pallas_call(
        _attn_kernel,
        grid=grid,
        in_specs=[
            pl.BlockSpec((1, REP, BQ, DH), lambda h, i: (h, 0, i, 0)),
            pl.BlockSpec((1, S, DH), lambda h, i: (h, 0, 0)),
            pl.BlockSpec((1, S, DH), lambda h, i: (h, 0, 0)),
        ],
        out_specs=pl.BlockSpec((1, REP, BQ, DH), lambda h, i: (h, 0, i, 0)),
        out_shape=jax.ShapeDtypeStruct((KVH, REP, S, DH), jnp.float32),
        scratch_shapes=[pltpu.VMEM((REP * BQ, S), jnp.float32)],
    )(q4, k3, v3)


def _proj_kernel(attn_ref, res_ref, wo_ref, w2_ref, rw_ref,
                 h_ref, x2_ref, lg_ref):
    h = res_ref[...] + _dot_t(attn_ref[...], wo_ref[...])
    h_ref[...] = h
    mu = jnp.mean(h, axis=1, keepdims=True)
    hc = h - mu
    var = jnp.mean(hc * hc, axis=1, keepdims=True)
    x2 = hc * jax.lax.rsqrt(var + 1e-5) * w2_ref[...]
    x2_ref[...] = x2
    lg_ref[...] = _dot_t(x2, rw_ref[...])


def _out_proj(attn_f, res, wo, ln2_w, router_w):
    grid = (S // BT,)
    return pl.pallas_call(
        _proj_kernel,
        grid=grid,
        in_specs=[
            pl.BlockSpec((BT, H * DH), lambda i: (i, 0)),
            pl.BlockSpec((BT, D), lambda i: (i, 0)),
            pl.BlockSpec((D, H * DH), lambda i: (0, 0)),
            pl.BlockSpec((1, D), lambda i: (0, 0)),
            pl.BlockSpec((E, D), lambda i: (0, 0)),
        ],
        out_specs=[
            pl.BlockSpec((BT, D), lambda i: (i, 0)),
            pl.BlockSpec((BT, D), lambda i: (i, 0)),
            pl.BlockSpec((BT, E), lambda i: (i, 0)),
        ],
        out_shape=[
            jax.ShapeDtypeStruct((S, D), jnp.float32),
            jax.ShapeDtypeStruct((S, D), jnp.float32),
            jax.ShapeDtypeStruct((S, E), jnp.float32),
        ],
    )(attn_f, res, wo, ln2_w.reshape(1, D), router_w)


def _route_kernel(lg_ref, dest_ref, topw_ref, cnt_ref):
    lg = lg_ref[...]
    m = jnp.max(lg, axis=1, keepdims=True)
    ex = jnp.exp(lg - m)
    probs = ex / jnp.sum(ex, axis=1, keepdims=True)
    iota = jax.lax.broadcasted_iota(jnp.int32, probs.shape, 1)
    m1 = jnp.max(probs, axis=1, keepdims=True)
    i1 = jnp.min(jnp.where(probs == m1, iota, E), axis=1, keepdims=True)
    masked = jnp.where(iota == i1, -1.0, probs)
    m2 = jnp.max(masked, axis=1, keepdims=True)
    i2 = jnp.min(jnp.where(masked == m2, iota, E), axis=1, keepdims=True)
    denom = m1 + m2
    topw_ref[...] = jnp.concatenate([m1 / denom, m2 / denom], axis=1)

    r = jax.lax.broadcasted_iota(jnp.int32, (RCH, RCH), 0)
    c = jax.lax.broadcasted_iota(jnp.int32, (RCH, RCH), 1)
    tril = (r > c).astype(jnp.float32)
    lane = jax.lax.broadcasted_iota(jnp.int32, (RCH, E), 1)
    carry = jnp.zeros((1, E), jnp.float32)
    nch = S // RCH
    for ch in range(2 * nch):
        if ch < nch:
            ecol = i1[ch * RCH:(ch + 1) * RCH]
        else:
            ecol = i2[(ch - nch) * RCH:(ch - nch + 1) * RCH]
        oh = (lane == ecol).astype(jnp.float32)
        ranks = jnp.dot(tril, oh,
                        preferred_element_type=jnp.float32) + carry
        rank = jnp.sum(jnp.where(lane == ecol, ranks, 0.0),
                       axis=1, keepdims=True)
        carry = carry + jnp.sum(oh, axis=0, keepdims=True)
        dst = rank + jnp.float32(CAP) * ecol.astype(jnp.float32)
        dest_ref[pl.ds(ch * RCH, RCH), :] = dst.astype(jnp.int32)
    cnt_ref[...] = carry.astype(jnp.int32)


def _routing(logits):
    return pl.pallas_call(
        _route_kernel,
        grid=(1,),
        in_specs=[pl.BlockSpec((S, E), lambda i: (0, 0))],
        out_specs=[
            pl.BlockSpec((NA, 1), lambda i: (0, 0)),
            pl.BlockSpec((S, TOPK), lambda i: (0, 0)),
            pl.BlockSpec((1, E), lambda i: (0, 0)),
        ],
        out_shape=[
            jax.ShapeDtypeStruct((NA, 1), jnp.int32),
            jax.ShapeDtypeStruct((S, TOPK), jnp.float32),
            jax.ShapeDtypeStruct((1, E), jnp.int32),
        ],
    )(logits)


_NW = 32
_CHUNK = NA // _NW


def _sc_permute(x2, tok, dest):
    mesh = plsc.VectorSubcoreMesh(core_axis_name="c", subcore_axis_name="s")

    @functools.partial(
        pl.kernel, mesh=mesh,
        out_type=jax.ShapeDtypeStruct((E * CAP, D), jnp.float32),
        scratch_types=[
            pltpu.VMEM((_CHUNK,), jnp.int32),
            pltpu.VMEM((_CHUNK,), jnp.int32),
            pltpu.VMEM((_CHUNK, D), jnp.float32),
            pltpu.SemaphoreType.DMA,
        ],
    )
    def k(x2_hbm, tok_hbm, dest_hbm, xs_hbm, tok_v, dest_v, rows_v, sem):
        wid = lax.axis_index("s") * 2 + lax.axis_index("c")
        base = wid * _CHUNK
        pltpu.sync_copy(tok_hbm.at[pl.ds(base, _CHUNK)], tok_v)
        pltpu.sync_copy(dest_hbm.at[pl.ds(base, _CHUNK)], dest_v)
        pltpu.async_copy(x2_hbm.at[tok_v], rows_v, sem).wait()
        pltpu.async_copy(rows_v, xs_hbm.at[dest_v], sem).wait()

    return k(x2, tok, dest)


def _sc_gather_back(ys, dest):
    mesh = plsc.VectorSubcoreMesh(core_axis_name="c", subcore_axis_name="s")

    @functools.partial(
        pl.kernel, mesh=mesh,
        out_type=jax.ShapeDtypeStruct((NA, D), jnp.float32),
        scratch_types=[
            pltpu.VMEM((_CHUNK,), jnp.int32),
            pltpu.VMEM((_CHUNK, D), jnp.float32),
            pltpu.SemaphoreType.DMA,
        ],
    )
    def k(ys_hbm, dest_hbm, yt_hbm, dest_v, rows_v, sem):
        wid = lax.axis_index("s") * 2 + lax.axis_index("c")
        base = wid * _CHUNK
        pltpu.sync_copy(dest_hbm.at[pl.ds(base, _CHUNK)], dest_v)
        pltpu.async_copy(ys_hbm.at[dest_v], rows_v, sem).wait()
        pltpu.sync_copy(rows_v, yt_hbm.at[pl.ds(base, _CHUNK)])

    return k(ys, dest)


def _gffn_kernel(cnt_ref, xs_ref, w1_ref, v1_ref, w2_ref, ys_ref, acc_ref):
    e = pl.program_id(0)
    f = pl.program_id(1)
    c = pl.program_id(2)
    cnt = cnt_ref[e]
    row = pl.multiple_of(c * BTC, BTC)

    @pl.when(c * BTC < cnt)
    def _():
        x = xs_ref[...]
        g = _dot_t(x, w1_ref[0])
        u = _dot_t(x, v1_ref[0])
        a = g * jax.lax.logistic(g) * u
        partial = _dot_t(a, w2_ref[0])

        @pl.when(f == 0)
        def _():
            acc_ref[pl.ds(row, BTC), :] = partial

        @pl.when(f > 0)
        def _():
            acc_ref[pl.ds(row, BTC), :] += partial

        @pl.when(f == FFB - 1)
        def _():
            ys_ref[...] = acc_ref[pl.ds(row, BTC), :]


def _clamped_block(e, c, cnt):
    nb = (cnt[e] + BTC - 1) // BTC
    cl = jnp.minimum(c, jnp.maximum(nb - 1, 0))
    return e * CAPB + cl


def _grouped_ffn(cnt, xs, w1, v1, w2):
    grid = (E, FFB, CAPB)
    return pl.pallas_call(
        _gffn_kernel,
        grid_spec=pltpu.PrefetchScalarGridSpec(
            num_scalar_prefetch=1,
            grid=grid,
            in_specs=[
                pl.BlockSpec((BTC, D),
                             lambda e, f, c, cnt: (_clamped_block(e, c, cnt), 0)),
                pl.BlockSpec((1, BF, D), lambda e, f, c, cnt: (e, f, 0)),
                pl.BlockSpec((1, BF, D), lambda e, f, c, cnt: (e, f, 0)),
                pl.BlockSpec((1, D, BF), lambda e, f, c, cnt: (e, 0, f)),
            ],
            out_specs=pl.BlockSpec(
                (BTC, D),
                lambda e, f, c, cnt: (_clamped_block(e, c, cnt), 0)),
            scratch_shapes=[pltpu.VMEM((CAP, D), jnp.float32)],
        ),
        out_shape=jax.ShapeDtypeStruct((E * CAP, D), jnp.float32),
    )(cnt, xs, w1, v1, w2)


def _combine_kernel(h_ref, y0_ref, y1_ref, tw_ref, o_ref):
    tw = tw_ref[...]
    o_ref[...] = (h_ref[...] + tw[:, 0:1] * y0_ref[...]
                  + tw[:, 1:2] * y1_ref[...])


def _combine(h, y0, y1, topw):
    grid = (S // BT,)
    return pl.pallas_call(
        _combine_kernel,
        grid=grid,
        in_specs=[
            pl.BlockSpec((BT, D), lambda i: (i, 0)),
            pl.BlockSpec((BT, D), lambda i: (i, 0)),
            pl.BlockSpec((BT, D), lambda i: (i, 0)),
            pl.BlockSpec((BT, TOPK), lambda i: (i, 0)),
        ],
        out_specs=pl.BlockSpec((BT, D), lambda i: (i, 0)),
        out_shape=jax.ShapeDtypeStruct((S, D), jnp.float32),
    )(h, y0, y1, topw)


def _rope_cos_sin(seq_len):
    inv_freq = 1.0 / (THETA ** (jnp.arange(0, DH, 2, dtype=jnp.float32) / DH))
    pos = jnp.arange(seq_len, dtype=jnp.float32)
    freqs = jnp.outer(pos, inv_freq)
    emb = jnp.concatenate([freqs, freqs], axis=-1)
    return jnp.cos(emb), jnp.sin(emb)


@jax.jit
def kernel(hidden_states, ln1_w, ln2_w, wqkv, wo, router_w, w1, v1, w2):
    x = hidden_states.reshape(S, D)
    cos, sin = _rope_cos_sin(S)
    qkv = _qkv_proj(x, ln1_w, wqkv, cos, sin)

    q4 = (qkv[:, :H * DH].reshape(S, H, DH).transpose(1, 0, 2)
          .reshape(KVH, REP, S, DH))
    k3 = qkv[:, H * DH:(H + KVH) * DH].reshape(S, KVH, DH).transpose(1, 0, 2)
    v3 = qkv[:, (H + KVH) * DH:].reshape(S, KVH, DH).transpose(1, 0, 2)

    attn = _attention(q4, k3, v3)
    attn_f = attn.transpose(2, 0, 1, 3).reshape(S, H * DH)

    h, x2, logits = _out_proj(attn_f, x, wo, ln2_w, router_w)

    dest2d, topw, cnt = _routing(logits)
    dest = dest2d.reshape(NA)
    tok = jnp.concatenate([jnp.arange(S, dtype=jnp.int32)] * TOPK)

    xs = _sc_permute(x2, tok, dest)
    ys = _grouped_ffn(cnt.reshape(E), xs, w1, v1, w2)
    yt = _sc_gather_back(ys, dest)

    out = _combine(h, yt[:S], yt[S:], topw)
    return out.reshape(B, S, D)


# speedup vs baseline: 1.2419x; 1.2419x over previous
import functools

import jax
import jax.numpy as jnp
from jax import lax
from jax.experimental import pallas as pl
from jax.experimental.pallas import tpu as pltpu
from jax.experimental.pallas import tpu_sc as plsc

B, S, D = 1, 2048, 768
H, KVH, DH = 12, 4, 64
E, TOPK, FF = 8, 2, 3072
THETA = 500000.0
CLIP = 8.0
REP = H // KVH

BT = 256
BQ = 512
BK = 512

NA = S * TOPK
CAP = S
BTC = 256
CAPB = CAP // BTC
BF = 1536
FFB = FF // BF
RCH = 512


def _dot_t(a, b):
    return jax.lax.dot_general(a, b, (((1,), (1,)), ((), ())),
                               preferred_element_type=jnp.float32)


def _qkv_kernel(x_ref, w_ref, wqkv_ref, cos_ref, sin_ref, o_ref):
    x = x_ref[...]
    mu = jnp.mean(x, axis=1, keepdims=True)
    xc = x - mu
    var = jnp.mean(xc * xc, axis=1, keepdims=True)
    xn = xc * jax.lax.rsqrt(var + 1e-5) * w_ref[...]
    qkv = _dot_t(xn, wqkv_ref[...])
    qkv = jnp.clip(qkv, -CLIP, CLIP)
    c = cos_ref[:, :DH // 2]
    s = sin_ref[:, :DH // 2]
    pieces = []
    for h in range(H + KVH):
        base = h * DH
        a = qkv[:, base:base + DH // 2]
        b = qkv[:, base + DH // 2:base + DH]
        pieces.append(a * c - b * s)
        pieces.append(b * c + a * s)
    pieces.append(qkv[:, (H + KVH) * DH:])
    o_ref[...] = jnp.concatenate(pieces, axis=1)


def _qkv_proj(x, ln1_w, wqkv, cos, sin):
    grid = (S // BT,)
    return pl.pallas_call(
        _qkv_kernel,
        grid=grid,
        in_specs=[
            pl.BlockSpec((BT, D), lambda i: (i, 0)),
            pl.BlockSpec((1, D), lambda i: (0, 0)),
            pl.BlockSpec(((H + 2 * KVH) * DH, D), lambda i: (0, 0)),
            pl.BlockSpec((BT, DH), lambda i: (i, 0)),
            pl.BlockSpec((BT, DH), lambda i: (i, 0)),
        ],
        out_specs=pl.BlockSpec((BT, (H + 2 * KVH) * DH), lambda i: (i, 0)),
        out_shape=jax.ShapeDtypeStruct((S, (H + 2 * KVH) * DH), jnp.float32),
    )(x, ln1_w.reshape(1, D), wqkv, cos, sin)


def _attn_kernel(q_ref, k_ref, v_ref, o_ref, s_ref):
    i = pl.program_id(1)
    nb = i + 1
    q = q_ref[0].reshape(REP * BQ, DH)
    scale = 1.0 / jnp.sqrt(jnp.float32(DH))

    def body1(j, m):
        k = k_ref[0, pl.ds(j * BK, BK), :]
        sc = _dot_t(q, k) * scale
        riota = jax.lax.broadcasted_iota(jnp.int32, (REP * BQ, BK), 0)
        qpos = i * BQ + (riota & (BQ - 1))
        kpos = j * BK + jax.lax.broadcasted_iota(jnp.int32, (REP * BQ, BK), 1)
        sc = jnp.where(qpos >= kpos, sc, -1e9)
        s_ref[:, pl.ds(j * BK, BK)] = sc
        return jnp.maximum(m, jnp.max(sc, axis=1, keepdims=True))

    m = jax.lax.fori_loop(0, nb, body1,
                          jnp.full((REP * BQ, 1), -jnp.inf, jnp.float32))

    def body2(j, carry):
        l, acc = carry
        pj = jnp.exp(s_ref[:, pl.ds(j * BK, BK)] - m)
        v = v_ref[0, pl.ds(j * BK, BK), :]
        return (l + jnp.sum(pj, axis=1, keepdims=True),
                acc + jnp.dot(pj, v, preferred_element_type=jnp.float32))

    l, acc = jax.lax.fori_loop(
        0, nb, body2,
        (jnp.zeros((REP * BQ, 1), jnp.float32),
         jnp.zeros((REP * BQ, DH), jnp.float32)))
    o_ref[0] = (acc / l).reshape(REP, BQ, DH)


def _attention(q4, k3, v3):
    grid = (KVH, S // BQ)
    return pl.pallas_call(
        _attn_kernel,
        grid=grid,
        in_specs=[
            pl.BlockSpec((1, REP, BQ, DH), lambda h, i: (h, 0, i, 0)),
            pl.BlockSpec((1, S, DH), lambda h, i: (h, 0, 0)),
            pl.BlockSpec((1, S, DH), lambda h, i: (h, 0, 0)),
        ],
        out_specs=pl.BlockSpec((1, REP, BQ, DH), lambda h, i: (h, 0, i, 0)),
        out_shape=jax.ShapeDtypeStruct((KVH, REP, S, DH), jnp.float32),
        scratch_shapes=[pltpu.VMEM((REP * BQ, S), jnp.float32)],
    )(q4, k3, v3)


def _proj_kernel(attn_ref, res_ref, wo_ref, w2_ref, rw_ref,
                 h_ref, x2_ref, lg_ref):
    h = res_ref[...] + _dot_t(attn_ref[...], wo_ref[...])
    h_ref[...] = h
    mu = jnp.mean(h, axis=1, keepdims=True)
    hc = h - mu
    var = jnp.mean(hc * hc, axis=1, keepdims=True)
    x2 = hc * jax.lax.rsqrt(var + 1e-5) * w2_ref[...]
    x2_ref[...] = x2
    lg_ref[...] = _dot_t(x2, rw_ref[...])


def _out_proj(attn_f, res, wo, ln2_w, router_w):
    grid = (S // BT,)
    return pl.pallas_call(
        _proj_kernel,
        grid=grid,
        in_specs=[
            pl.BlockSpec((BT, H * DH), lambda i: (i, 0)),
            pl.BlockSpec((BT, D), lambda i: (i, 0)),
            pl.BlockSpec((D, H * DH), lambda i: (0, 0)),
            pl.BlockSpec((1, D), lambda i: (0, 0)),
            pl.BlockSpec((E, D), lambda i: (0, 0)),
        ],
        out_specs=[
            pl.BlockSpec((BT, D), lambda i: (i, 0)),
            pl.BlockSpec((BT, D), lambda i: (i, 0)),
            pl.BlockSpec((BT, E), lambda i: (i, 0)),
        ],
        out_shape=[
            jax.ShapeDtypeStruct((S, D), jnp.float32),
            jax.ShapeDtypeStruct((S, D), jnp.float32),
            jax.ShapeDtypeStruct((S, E), jnp.float32),
        ],
    )(attn_f, res, wo, ln2_w.reshape(1, D), router_w)


def _route_kernel(lg_ref, dest_ref, topw_ref, cnt_ref):
    lg = lg_ref[...]
    m = jnp.max(lg, axis=1, keepdims=True)
    ex = jnp.exp(lg - m)
    probs = ex / jnp.sum(ex, axis=1, keepdims=True)
    iota = jax.lax.broadcasted_iota(jnp.int32, probs.shape, 1)
    m1 = jnp.max(probs, axis=1, keepdims=True)
    i1 = jnp.min(jnp.where(probs == m1, iota, E), axis=1, keepdims=True)
    masked = jnp.where(iota == i1, -1.0, probs)
    m2 = jnp.max(masked, axis=1, keepdims=True)
    i2 = jnp.min(jnp.where(masked == m2, iota, E), axis=1, keepdims=True)
    denom = m1 + m2
    topw_ref[...] = jnp.concatenate([m1 / denom, m2 / denom], axis=1)

    r = jax.lax.broadcasted_iota(jnp.int32, (RCH, RCH), 0)
    c = jax.lax.broadcasted_iota(jnp.int32, (RCH, RCH), 1)
    tril = (r > c).astype(jnp.float32)
    lane = jax.lax.broadcasted_iota(jnp.int32, (RCH, E), 1)
    carry = jnp.zeros((1, E), jnp.float32)
    nch = S // RCH
    for ch in range(2 * nch):
        if ch < nch:
            ecol = i1[ch * RCH:(ch + 1) * RCH]
        else:
            ecol = i2[(ch - nch) * RCH:(ch - nch + 1) * RCH]
        oh = (lane == ecol).astype(jnp.float32)
        ranks = jnp.dot(tril, oh,
                        preferred_element_type=jnp.float32) + carry
        rank = jnp.sum(jnp.where(lane == ecol, ranks, 0.0),
                       axis=1, keepdims=True)
        carry = carry + jnp.sum(oh, axis=0, keepdims=True)
        dst = rank + jnp.float32(CAP) * ecol.astype(jnp.float32)
        dest_ref[pl.ds(ch * RCH, RCH), :] = dst.astype(jnp.int32)
    cnt_ref[...] = carry.astype(jnp.int32)


def _routing(logits):
    return pl.pallas_call(
        _route_kernel,
        grid=(1,),
        in_specs=[pl.BlockSpec((S, E), lambda i: (0, 0))],
        out_specs=[
            pl.BlockSpec((NA, 1), lambda i: (0, 0)),
            pl.BlockSpec((S, TOPK), lambda i: (0, 0)),
            pl.BlockSpec((1, E), lambda i: (0, 0)),
        ],
        out_shape=[
            jax.ShapeDtypeStruct((NA, 1), jnp.int32),
            jax.ShapeDtypeStruct((S, TOPK), jnp.float32),
            jax.ShapeDtypeStruct((1, E), jnp.int32),
        ],
    )(logits)


_NW = 32
_CHUNK = NA // _NW


def _sc_permute(x2, tok, dest):
    mesh = plsc.VectorSubcoreMesh(core_axis_name="c", subcore_axis_name="s")

    @functools.partial(
        pl.kernel, mesh=mesh,
        out_type=jax.ShapeDtypeStruct((E * CAP, D), jnp.float32),
        scratch_types=[
            pltpu.VMEM((_CHUNK,), jnp.int32),
            pltpu.VMEM((_CHUNK,), jnp.int32),
            pltpu.VMEM((_CHUNK, D), jnp.float32),
            pltpu.SemaphoreType.DMA,
        ],
    )
    def k(x2_hbm, tok_hbm, dest_hbm, xs_hbm, tok_v, dest_v, rows_v, sem):
        wid = lax.axis_index("s") * 2 + lax.axis_index("c")
        base = wid * _CHUNK
        pltpu.sync_copy(tok_hbm.at[pl.ds(base, _CHUNK)], tok_v)
        pltpu.sync_copy(dest_hbm.at[pl.ds(base, _CHUNK)], dest_v)
        pltpu.async_copy(x2_hbm.at[tok_v], rows_v, sem).wait()
        pltpu.async_copy(rows_v, xs_hbm.at[dest_v], sem).wait()

    return k(x2, tok, dest)


def _sc_gather_back(ys, dest):
    mesh = plsc.VectorSubcoreMesh(core_axis_name="c", subcore_axis_name="s")

    @functools.partial(
        pl.kernel, mesh=mesh,
        out_type=jax.ShapeDtypeStruct((NA, D), jnp.float32),
        scratch_types=[
            pltpu.VMEM((_CHUNK,), jnp.int32),
            pltpu.VMEM((_CHUNK, D), jnp.float32),
            pltpu.SemaphoreType.DMA,
        ],
    )
    def k(ys_hbm, dest_hbm, yt_hbm, dest_v, rows_v, sem):
        wid = lax.axis_index("s") * 2 + lax.axis_index("c")
        base = wid * _CHUNK
        pltpu.sync_copy(dest_hbm.at[pl.ds(base, _CHUNK)], dest_v)
        pltpu.async_copy(ys_hbm.at[dest_v], rows_v, sem).wait()
        pltpu.sync_copy(rows_v, yt_hbm.at[pl.ds(base, _CHUNK)])

    return k(ys, dest)


def _gffn_kernel(cnt_ref, xs_ref, w1_ref, v1_ref, w2_ref, ys_ref, acc_ref):
    e = pl.program_id(0)
    f = pl.program_id(1)
    c = pl.program_id(2)
    cnt = cnt_ref[e]
    row = pl.multiple_of(c * BTC, BTC)

    @pl.when(c * BTC < cnt)
    def _():
        x = xs_ref[...]
        g = _dot_t(x, w1_ref[0])
        u = _dot_t(x, v1_ref[0])
        a = g * jax.lax.logistic(g) * u
        partial = _dot_t(a, w2_ref[0])

        @pl.when(f == 0)
        def _():
            acc_ref[pl.ds(row, BTC), :] = partial

        @pl.when(f > 0)
        def _():
            acc_ref[pl.ds(row, BTC), :] += partial

        @pl.when(f == FFB - 1)
        def _():
            ys_ref[...] = acc_ref[pl.ds(row, BTC), :]


def _clamped_block(e, c, cnt):
    nb = (cnt[e] + BTC - 1) // BTC
    cl = jnp.minimum(c, jnp.maximum(nb - 1, 0))
    return e * CAPB + cl


def _grouped_ffn(cnt, xs, w1, v1, w2):
    grid = (E, FFB, CAPB)
    return pl.pallas_call(
        _gffn_kernel,
        grid_spec=pltpu.PrefetchScalarGridSpec(
            num_scalar_prefetch=1,
            grid=grid,
            in_specs=[
                pl.BlockSpec((BTC, D),
                             lambda e, f, c, cnt: (_clamped_block(e, c, cnt), 0)),
                pl.BlockSpec((1, BF, D), lambda e, f, c, cnt: (e, f, 0)),
                pl.BlockSpec((1, BF, D), lambda e, f, c, cnt: (e, f, 0)),
                pl.BlockSpec((1, D, BF), lambda e, f, c, cnt: (e, 0, f)),
            ],
            out_specs=pl.BlockSpec(
                (BTC, D),
                lambda e, f, c, cnt: (_clamped_block(e, c, cnt), 0)),
            scratch_shapes=[pltpu.VMEM((CAP, D), jnp.float32)],
        ),
        out_shape=jax.ShapeDtypeStruct((E * CAP, D), jnp.float32),
    )(cnt, xs, w1, v1, w2)


def _combine_kernel(h_ref, y0_ref, y1_ref, tw_ref, o_ref):
    tw = tw_ref[...]
    o_ref[...] = (h_ref[...] + tw[:, 0:1] * y0_ref[...]
                  + tw[:, 1:2] * y1_ref[...])


def _combine(h, y0, y1, topw):
    grid = (S // BT,)
    return pl.pallas_call(
        _combine_kernel,
        grid=grid,
        in_specs=[
            pl.BlockSpec((BT, D), lambda i: (i, 0)),
            pl.BlockSpec((BT, D), lambda i: (i, 0)),
            pl.BlockSpec((BT, D), lambda i: (i, 0)),
            pl.BlockSpec((BT, TOPK), lambda i: (i, 0)),
        ],
        out_specs=pl.BlockSpec((BT, D), lambda i: (i, 0)),
        out_shape=jax.ShapeDtypeStruct((S, D), jnp.float32),
    )(h, y0, y1, topw)


def _rope_cos_sin(seq_len):
    inv_freq = 1.0 / (THETA ** (jnp.arange(0, DH, 2, dtype=jnp.float32) / DH))
    pos = jnp.arange(seq_len, dtype=jnp.float32)
    freqs = jnp.outer(pos, inv_freq)
    emb = jnp.concatenate([freqs, freqs], axis=-1)
    return jnp.cos(emb), jnp.sin(emb)


@jax.jit
def kernel(hidden_states, ln1_w, ln2_w, wqkv, wo, router_w, w1, v1, w2):
    x = hidden_states.reshape(S, D)
    cos, sin = _rope_cos_sin(S)
    qkv = _qkv_proj(x, ln1_w, wqkv, cos, sin)

    q4 = (qkv[:, :H * DH].reshape(S, H, DH).transpose(1, 0, 2)
          .reshape(KVH, REP, S, DH))
    k3 = qkv[:, H * DH:(H + KVH) * DH].reshape(S, KVH, DH).transpose(1, 0, 2)
    v3 = qkv[:, (H + KVH) * DH:].reshape(S, KVH, DH).transpose(1, 0, 2)

    attn = _attention(q4, k3, v3)
    attn_f = attn.transpose(2, 0, 1, 3).reshape(S, H * DH)

    h, x2, logits = _out_proj(attn_f, x, wo, ln2_w, router_w)

    dest2d, topw, cnt = _routing(logits)
    dest = dest2d.reshape(NA)
    tok = jnp.concatenate([jnp.arange(S, dtype=jnp.int32)] * TOPK)

    xs = _sc_permute(x2, tok, dest)
    ys = _grouped_ffn(cnt.reshape(E), xs, w1, v1, w2)
    yt = _sc_gather_back(ys, dest)

    out = _combine(h, yt[:S], yt[S:], topw)
    return out.reshape(B, S, D)


# speedup vs baseline: 1.4005x; 1.1277x over previous
import functools

import numpy as np

import jax
import jax.numpy as jnp
from jax import lax
from jax.experimental import pallas as pl
from jax.experimental.pallas import tpu as pltpu
from jax.experimental.pallas import tpu_sc as plsc

B, S, D = 1, 2048, 768
H, KVH, DH = 12, 4, 64
E, TOPK, FF = 8, 2, 3072
THETA = 500000.0
CLIP = 8.0
REP = H // KVH

BT = 256
BQ = 512
BK = 512

NA = S * TOPK
CAP = S
BTC = 256
CAPB = CAP // BTC
BF = 1536
FFB = FF // BF
RCH = 512


def _dot_t(a, b):
    return jax.lax.dot_general(a, b, (((1,), (1,)), ((), ())),
                               preferred_element_type=jnp.float32)


def _qkv_kernel(x_ref, w_ref, wqkv_ref, cos_ref, sin_ref,
                q_ref, k_ref, v_ref):
    x = x_ref[...]
    mu = jnp.mean(x, axis=1, keepdims=True)
    xc = x - mu
    var = jnp.mean(xc * xc, axis=1, keepdims=True)
    xn = xc * jax.lax.rsqrt(var + 1e-5) * w_ref[...]
    qkv = _dot_t(xn, wqkv_ref[...])
    qkv = jnp.clip(qkv, -CLIP, CLIP)
    c = cos_ref[:, :DH // 2]
    s = sin_ref[:, :DH // 2]

    def rope(h):
        base = h * DH
        a = qkv[:, base:base + DH // 2]
        b = qkv[:, base + DH // 2:base + DH]
        return jnp.concatenate([a * c - b * s, b * c + a * s], axis=1)

    for h in range(H):
        q_ref[h // REP, h % REP, :, :] = rope(h)
    for g in range(KVH):
        k_ref[g, :, :] = rope(H + g)
        v_ref[g, :, :] = qkv[:, (H + KVH + g) * DH:(H + KVH + g + 1) * DH]


def _qkv_proj(x, ln1_w, wqkv, cos, sin):
    grid = (S // BT,)
    return pl.pallas_call(
        _qkv_kernel,
        grid=grid,
        in_specs=[
            pl.BlockSpec((BT, D), lambda i: (i, 0)),
            pl.BlockSpec((1, D), lambda i: (0, 0)),
            pl.BlockSpec(((H + 2 * KVH) * DH, D), lambda i: (0, 0)),
            pl.BlockSpec((BT, DH), lambda i: (i, 0)),
            pl.BlockSpec((BT, DH), lambda i: (i, 0)),
        ],
        out_specs=[
            pl.BlockSpec((KVH, REP, BT, DH), lambda i: (0, 0, i, 0)),
            pl.BlockSpec((KVH, BT, DH), lambda i: (0, i, 0)),
            pl.BlockSpec((KVH, BT, DH), lambda i: (0, i, 0)),
        ],
        out_shape=[
            jax.ShapeDtypeStruct((KVH, REP, S, DH), jnp.float32),
            jax.ShapeDtypeStruct((KVH, S, DH), jnp.float32),
            jax.ShapeDtypeStruct((KVH, S, DH), jnp.float32),
        ],
    )(x, ln1_w.reshape(1, D), wqkv, cos, sin)


def _attn_kernel(q_ref, k_ref, v_ref, o_ref, s_ref):
    i = pl.program_id(0)
    nb = i + 1
    scale = 1.0 / jnp.sqrt(jnp.float32(DH))

    for g in range(KVH):
        q = q_ref[g, :, :, :].reshape(REP * BQ, DH)

        def body1(j, m):
            k = k_ref[g, pl.ds(j * BK, BK), :]
            sc = _dot_t(q, k) * scale
            riota = jax.lax.broadcasted_iota(jnp.int32, (REP * BQ, BK), 0)
            qpos = i * BQ + (riota & (BQ - 1))
            kpos = j * BK + jax.lax.broadcasted_iota(
                jnp.int32, (REP * BQ, BK), 1)
            sc = jnp.where(qpos >= kpos, sc, -1e9)
            s_ref[:, pl.ds(j * BK, BK)] = sc
            return jnp.maximum(m, jnp.max(sc, axis=1, keepdims=True))

        m = jax.lax.fori_loop(0, nb, body1,
                              jnp.full((REP * BQ, 1), -jnp.inf, jnp.float32))

        def body2(j, carry):
            l, acc = carry
            pj = jnp.exp(s_ref[:, pl.ds(j * BK, BK)] - m)
            v = v_ref[g, pl.ds(j * BK, BK), :]
            return (l + jnp.sum(pj, axis=1, keepdims=True),
                    acc + jnp.dot(pj, v, preferred_element_type=jnp.float32))

        l, acc = jax.lax.fori_loop(
            0, nb, body2,
            (jnp.zeros((REP * BQ, 1), jnp.float32),
             jnp.zeros((REP * BQ, DH), jnp.float32)))
        res = acc / l
        for r in range(REP):
            h = g * REP + r
            o_ref[:, h * DH:(h + 1) * DH] = res[r * BQ:(r + 1) * BQ, :]


def _attention(q4, k3, v3):
    grid = (S // BQ,)
    return pl.pallas_call(
        _attn_kernel,
        grid=grid,
        in_specs=[
            pl.BlockSpec((KVH, REP, BQ, DH), lambda i: (0, 0, i, 0)),
            pl.BlockSpec((KVH, S, DH), lambda i: (0, 0, 0)),
            pl.BlockSpec((KVH, S, DH), lambda i: (0, 0, 0)),
        ],
        out_specs=pl.BlockSpec((BQ, H * DH), lambda i: (i, 0)),
        out_shape=jax.ShapeDtypeStruct((S, H * DH), jnp.float32),
        scratch_shapes=[pltpu.VMEM((REP * BQ, S), jnp.float32)],
    )(q4, k3, v3)


def _proj_kernel(attn_ref, res_ref, wo_ref, w2_ref, rw_ref,
                 h_ref, x2_ref, lg_ref):
    h = res_ref[...] + _dot_t(attn_ref[...], wo_ref[...])
    h_ref[...] = h
    mu = jnp.mean(h, axis=1, keepdims=True)
    hc = h - mu
    var = jnp.mean(hc * hc, axis=1, keepdims=True)
    x2 = hc * jax.lax.rsqrt(var + 1e-5) * w2_ref[...]
    x2_ref[...] = x2
    lg_ref[...] = _dot_t(x2, rw_ref[...])


def _out_proj(attn_f, res, wo, ln2_w, router_w):
    grid = (S // BT,)
    return pl.pallas_call(
        _proj_kernel,
        grid=grid,
        in_specs=[
            pl.BlockSpec((BT, H * DH), lambda i: (i, 0)),
            pl.BlockSpec((BT, D), lambda i: (i, 0)),
            pl.BlockSpec((D, H * DH), lambda i: (0, 0)),
            pl.BlockSpec((1, D), lambda i: (0, 0)),
            pl.BlockSpec((E, D), lambda i: (0, 0)),
        ],
        out_specs=[
            pl.BlockSpec((BT, D), lambda i: (i, 0)),
            pl.BlockSpec((BT, D), lambda i: (i, 0)),
            pl.BlockSpec((BT, E), lambda i: (i, 0)),
        ],
        out_shape=[
            jax.ShapeDtypeStruct((S, D), jnp.float32),
            jax.ShapeDtypeStruct((S, D), jnp.float32),
            jax.ShapeDtypeStruct((S, E), jnp.float32),
        ],
    )(attn_f, res, wo, ln2_w.reshape(1, D), router_w)


def _route_kernel(lg_ref, dest_ref, topw_ref, cnt_ref):
    lg = lg_ref[...]
    m = jnp.max(lg, axis=1, keepdims=True)
    ex = jnp.exp(lg - m)
    probs = ex / jnp.sum(ex, axis=1, keepdims=True)
    iota = jax.lax.broadcasted_iota(jnp.int32, probs.shape, 1)
    m1 = jnp.max(probs, axis=1, keepdims=True)
    i1 = jnp.min(jnp.where(probs == m1, iota, E), axis=1, keepdims=True)
    masked = jnp.where(iota == i1, -1.0, probs)
    m2 = jnp.max(masked, axis=1, keepdims=True)
    i2 = jnp.min(jnp.where(masked == m2, iota, E), axis=1, keepdims=True)
    denom = m1 + m2
    topw_ref[...] = jnp.concatenate([m1 / denom, m2 / denom], axis=1)

    r = jax.lax.broadcasted_iota(jnp.int32, (RCH, RCH), 0)
    c = jax.lax.broadcasted_iota(jnp.int32, (RCH, RCH), 1)
    tril = (r > c).astype(jnp.float32)
    lane = jax.lax.broadcasted_iota(jnp.int32, (RCH, E), 1)
    carry = jnp.zeros((1, E), jnp.float32)
    nch = S // RCH
    for ch in range(2 * nch):
        if ch < nch:
            ecol = i1[ch * RCH:(ch + 1) * RCH]
        else:
            ecol = i2[(ch - nch) * RCH:(ch - nch + 1) * RCH]
        oh = (lane == ecol).astype(jnp.float32)
        ranks = jnp.dot(tril, oh,
                        preferred_element_type=jnp.float32) + carry
        rank = jnp.sum(jnp.where(lane == ecol, ranks, 0.0),
                       axis=1, keepdims=True)
        carry = carry + jnp.sum(oh, axis=0, keepdims=True)
        dst = rank + jnp.float32(CAP) * ecol.astype(jnp.float32)
        dest_ref[pl.ds(ch * RCH, RCH), :] = dst.astype(jnp.int32)
    cnt_ref[...] = carry.astype(jnp.int32)


def _routing(logits):
    return pl.pallas_call(
        _route_kernel,
        grid=(1,),
        in_specs=[pl.BlockSpec((S, E), lambda i: (0, 0))],
        out_specs=[
            pl.BlockSpec((NA, 1), lambda i: (0, 0)),
            pl.BlockSpec((S, TOPK), lambda i: (0, 0)),
            pl.BlockSpec((1, E), lambda i: (0, 0)),
        ],
        out_shape=[
            jax.ShapeDtypeStruct((NA, 1), jnp.int32),
            jax.ShapeDtypeStruct((S, TOPK), jnp.float32),
            jax.ShapeDtypeStruct((1, E), jnp.int32),
        ],
    )(logits)


_NW = 32
_CHUNK = NA // _NW


def _sc_permute(x2, tok, dest):
    mesh = plsc.VectorSubcoreMesh(core_axis_name="c", subcore_axis_name="s")

    @functools.partial(
        pl.kernel, mesh=mesh,
        out_type=jax.ShapeDtypeStruct((E * CAP, D), jnp.float32),
        scratch_types=[
            pltpu.VMEM((_CHUNK,), jnp.int32),
            pltpu.VMEM((_CHUNK,), jnp.int32),
            pltpu.VMEM((_CHUNK, D), jnp.float32),
            pltpu.SemaphoreType.DMA,
        ],
    )
    def k(x2_hbm, tok_hbm, dest_hbm, xs_hbm, tok_v, dest_v, rows_v, sem):
        wid = lax.axis_index("s") * 2 + lax.axis_index("c")
        base = wid * _CHUNK
        pltpu.sync_copy(tok_hbm.at[pl.ds(base, _CHUNK)], tok_v)
        pltpu.sync_copy(dest_hbm.at[pl.ds(base, _CHUNK)], dest_v)
        pltpu.async_copy(x2_hbm.at[tok_v], rows_v, sem).wait()
        pltpu.async_copy(rows_v, xs_hbm.at[dest_v], sem).wait()

    return k(x2, tok, dest)


def _sc_gather_back(ys, dest):
    mesh = plsc.VectorSubcoreMesh(core_axis_name="c", subcore_axis_name="s")

    @functools.partial(
        pl.kernel, mesh=mesh,
        out_type=jax.ShapeDtypeStruct((NA, D), jnp.float32),
        scratch_types=[
            pltpu.VMEM((_CHUNK,), jnp.int32),
            pltpu.VMEM((_CHUNK, D), jnp.float32),
            pltpu.SemaphoreType.DMA,
        ],
    )
    def k(ys_hbm, dest_hbm, yt_hbm, dest_v, rows_v, sem):
        wid = lax.axis_index("s") * 2 + lax.axis_index("c")
        base = wid * _CHUNK
        pltpu.sync_copy(dest_hbm.at[pl.ds(base, _CHUNK)], dest_v)
        pltpu.async_copy(ys_hbm.at[dest_v], rows_v, sem).wait()
        pltpu.sync_copy(rows_v, yt_hbm.at[pl.ds(base, _CHUNK)])

    return k(ys, dest)


def _gffn_kernel(cnt_ref, xs_ref, w1_ref, v1_ref, w2_ref, ys_ref, acc_ref):
    e = pl.program_id(0)
    f = pl.program_id(1)
    c = pl.program_id(2)
    cnt = cnt_ref[e]
    row = pl.multiple_of(c * BTC, BTC)

    @pl.when(c * BTC < cnt)
    def _():
        x = xs_ref[...]
        g = _dot_t(x, w1_ref[0])
        u = _dot_t(x, v1_ref[0])
        a = g * jax.lax.logistic(g) * u
        partial = _dot_t(a, w2_ref[0])

        @pl.when(f == 0)
        def _():
            acc_ref[pl.ds(row, BTC), :] = partial

        @pl.when(f > 0)
        def _():
            acc_ref[pl.ds(row, BTC), :] += partial

        @pl.when(f == FFB - 1)
        def _():
            ys_ref[...] = acc_ref[pl.ds(row, BTC), :]


def _clamped_block(e, c, cnt):
    nb = (cnt[e] + BTC - 1) // BTC
    cl = jnp.minimum(c, jnp.maximum(nb - 1, 0))
    return e * CAPB + cl


def _grouped_ffn(cnt, xs, w1, v1, w2):
    grid = (E, FFB, CAPB)
    return pl.pallas_call(
        _gffn_kernel,
        grid_spec=pltpu.PrefetchScalarGridSpec(
            num_scalar_prefetch=1,
            grid=grid,
            in_specs=[
                pl.BlockSpec((BTC, D),
                             lambda e, f, c, cnt: (_clamped_block(e, c, cnt), 0)),
                pl.BlockSpec((1, BF, D), lambda e, f, c, cnt: (e, f, 0)),
                pl.BlockSpec((1, BF, D), lambda e, f, c, cnt: (e, f, 0)),
                pl.BlockSpec((1, D, BF), lambda e, f, c, cnt: (e, 0, f)),
            ],
            out_specs=pl.BlockSpec(
                (BTC, D),
                lambda e, f, c, cnt: (_clamped_block(e, c, cnt), 0)),
            scratch_shapes=[pltpu.VMEM((CAP, D), jnp.float32)],
        ),
        out_shape=jax.ShapeDtypeStruct((E * CAP, D), jnp.float32),
    )(cnt, xs, w1, v1, w2)


def _combine_kernel(h_ref, y0_ref, y1_ref, tw_ref, o_ref):
    tw = tw_ref[...]
    o_ref[...] = (h_ref[...] + tw[:, 0:1] * y0_ref[...]
                  + tw[:, 1:2] * y1_ref[...])


def _combine(h, yt, topw):
    grid = (S // BT,)
    return pl.pallas_call(
        _combine_kernel,
        grid=grid,
        in_specs=[
            pl.BlockSpec((BT, D), lambda i: (i, 0)),
            pl.BlockSpec((BT, D), lambda i: (i, 0)),
            pl.BlockSpec((BT, D), lambda i: (i + S // BT, 0)),
            pl.BlockSpec((BT, TOPK), lambda i: (i, 0)),
        ],
        out_specs=pl.BlockSpec((BT, D), lambda i: (i, 0)),
        out_shape=jax.ShapeDtypeStruct((S, D), jnp.float32),
    )(h, yt, yt, topw)


def _np_rope_cos_sin(seq_len):
    inv_freq = 1.0 / (THETA ** (np.arange(0, DH, 2, dtype=np.float32) / DH))
    pos = np.arange(seq_len, dtype=np.float32)
    freqs = np.outer(pos, inv_freq)
    emb = np.concatenate([freqs, freqs], axis=-1)
    return np.cos(emb).astype(np.float32), np.sin(emb).astype(np.float32)


_COS_NP, _SIN_NP = _np_rope_cos_sin(S)
_TOK_NP = np.concatenate([np.arange(S, dtype=np.int32)] * TOPK)


@jax.jit
def kernel(hidden_states, ln1_w, ln2_w, wqkv, wo, router_w, w1, v1, w2):
    x = hidden_states.reshape(S, D)
    cos = jnp.asarray(_COS_NP)
    sin = jnp.asarray(_SIN_NP)
    q4, k3, v3 = _qkv_proj(x, ln1_w, wqkv, cos, sin)

    attn_f = _attention(q4, k3, v3)

    h, x2, logits = _out_proj(attn_f, x, wo, ln2_w, router_w)

    dest2d, topw, cnt = _routing(logits)
    dest = dest2d.reshape(NA)
    tok = jnp.asarray(_TOK_NP)

    xs = _sc_permute(x2, tok, dest)
    ys = _grouped_ffn(cnt.reshape(E), xs, w1, v1, w2)
    yt = _sc_gather_back(ys, dest)

    out = _combine(h, yt, topw)
    return out.reshape(B, S, D)
